# baseline (speedup 1.0000x reference)
import jax
import jax.numpy as jnp
from jax import lax
from jax.experimental import pallas as pl
from jax.experimental.pallas import tpu as pltpu
from jax.experimental.pallas import tpu_sc as plsc

_NUM_GRAPHS = 64
_EPS = 1e-05
_BLOCK = 4000
_N = 100000
_F = 512
_SC_CORES = 2
_SC_SUBCORES = 16
_CHUNK = 128
_FGROUPS = _F // 16


def _sc_stats_body(x_hbm, bnd_hbm, sums_hbm, sq_hbm, bnd_v, xbuf,
                   acc_s, acc_q):
    c = lax.axis_index("c")
    s = lax.axis_index("s")
    w = s * _SC_CORES + c
    pltpu.sync_copy(bnd_hbm, bnd_v)
    zeros = jnp.zeros((16,), jnp.float32)
    for g in range(2):
        for fg in range(_FGROUPS):
            acc_s[g, pl.ds(fg * 16, 16)] = zeros
            acc_q[g, pl.ds(fg * 16, 16)] = zeros
    bv = bnd_v[pl.ds(2 * w, 16)]
    s0 = bv[0]
    s1 = bv[1]
    s2 = bv[2]
    s0a = (s0 // 8) * 8
    nch = (s2 - s0a + _CHUNK - 1) // _CHUNK

    def chunk_body(ci, _):
        r0 = s0a + ci * _CHUNK
        r0c = jnp.minimum(r0, _N - _CHUNK)
        pltpu.sync_copy(x_hbm.at[pl.ds(r0c, _CHUNK)], xbuf)
        for seg in range(2):
            lo = jnp.maximum(r0, s0 if seg == 0 else s1) - r0c
            hi = jnp.minimum(r0 + _CHUNK, s1 if seg == 0 else s2) - r0c
            for fg in range(_FGROUPS):
                def row_body(r, carry, fg=fg):
                    vs, vq = carry
                    v = xbuf[r, pl.ds(fg * 16, 16)]
                    return vs + v, vq + v * v

                vs, vq = plsc.parallel_loop(
                    lo, hi, unroll=4, carry=(zeros, zeros))(row_body)
                acc_s[seg, pl.ds(fg * 16, 16)] += vs
                acc_q[seg, pl.ds(fg * 16, 16)] += vq
        return 0

    lax.fori_loop(0, nch, chunk_body, 0)
    pltpu.sync_copy(acc_s, sums_hbm.at[pl.ds(8 * w, 8)])
    pltpu.sync_copy(acc_q, sq_hbm.at[pl.ds(8 * w, 8)])


def _sc_stats(x, bounds, interpret=False):
    mesh = plsc.VectorSubcoreMesh(
        core_axis_name="c", subcore_axis_name="s",
        num_cores=_SC_CORES, num_subcores=_SC_SUBCORES)
    f = x.shape[1]
    return pl.kernel(
        _sc_stats_body,
        out_type=[
            jax.ShapeDtypeStruct((8 * _SC_CORES * _SC_SUBCORES, f), jnp.float32),
            jax.ShapeDtypeStruct((8 * _SC_CORES * _SC_SUBCORES, f), jnp.float32),
        ],
        mesh=mesh,
        scratch_types=[
            pltpu.VMEM((80,), jnp.int32),
            pltpu.VMEM((_CHUNK, f), jnp.float32),
            pltpu.VMEM((8, f), jnp.float32),
            pltpu.VMEM((8, f), jnp.float32),
        ],
        interpret=interpret,
    )(x, bounds)


def _norm_kernel(batch_ref, x_ref, sums_ref, sq_ref, cnt_ref, w_ref, bias_ref,
                 out_ref):
    b = batch_ref[0, 0, :]
    sums = sums_ref[...]
    sq = sq_ref[...]
    cnt = cnt_ref[:, :1]
    mean = sums / jnp.maximum(cnt, 1.0)
    var = (sq - sums * mean) / jnp.maximum(cnt - 1.0, 1.0)
    var = jnp.maximum(var, 0.0)
    scale = w_ref[...] / (jnp.sqrt(var) + _EPS)
    offset = bias_ref[...] - mean * scale
    gids = jax.lax.broadcasted_iota(jnp.int32, (_BLOCK, _NUM_GRAPHS), 1)
    oh = (gids == b[:, None]).astype(jnp.bfloat16)
    tbl = jnp.concatenate([scale, offset], axis=1)
    hi = tbl.astype(jnp.bfloat16)
    lo = (tbl - hi.astype(jnp.float32)).astype(jnp.bfloat16)
    g_hi = jax.lax.dot(oh, hi, preferred_element_type=jnp.float32)
    g_lo = jax.lax.dot(oh, lo, preferred_element_type=jnp.float32)
    g = g_hi + g_lo
    f = x_ref.shape[1]
    out_ref[...] = x_ref[...] * g[:, :f] + g[:, f:]


def _impl(x, batch, weight, bias, interpret=False):
    n, f = x.shape
    nblk = n // _BLOCK
    batch_r = batch.reshape(nblk, 1, _BLOCK)
    w2 = weight.reshape(1, f)
    b2 = bias.reshape(1, f)

    gid = jnp.arange(_NUM_GRAPHS, dtype=jnp.int32)
    starts = jnp.searchsorted(batch, gid).astype(jnp.int32)
    bounds = jnp.concatenate(
        [starts, jnp.full((80 - _NUM_GRAPHS,), n, jnp.int32)])
    cnt = (bounds[1:_NUM_GRAPHS + 1] - bounds[:_NUM_GRAPHS]).astype(jnp.float32)
    cnt128 = jnp.broadcast_to(cnt[:, None], (_NUM_GRAPHS, 128))

    sums_big, sq_big = _sc_stats(x, bounds, interpret=interpret)
    nw = _SC_CORES * _SC_SUBCORES
    sums = sums_big.reshape(nw, 8, f)[:, :2, :].reshape(_NUM_GRAPHS, f)
    sq = sq_big.reshape(nw, 8, f)[:, :2, :].reshape(_NUM_GRAPHS, f)

    out = pl.pallas_call(
        _norm_kernel,
        grid=(nblk,),
        in_specs=[
            pl.BlockSpec((1, 1, _BLOCK), lambda i: (i, 0, 0)),
            pl.BlockSpec((_BLOCK, f), lambda i: (i, 0)),
            pl.BlockSpec((_NUM_GRAPHS, f), lambda i: (0, 0)),
            pl.BlockSpec((_NUM_GRAPHS, f), lambda i: (0, 0)),
            pl.BlockSpec((_NUM_GRAPHS, 128), lambda i: (0, 0)),
            pl.BlockSpec((1, f), lambda i: (0, 0)),
            pl.BlockSpec((1, f), lambda i: (0, 0)),
        ],
        out_specs=pl.BlockSpec((_BLOCK, f), lambda i: (i, 0)),
        out_shape=jax.ShapeDtypeStruct((n, f), jnp.float32),
        interpret=interpret,
    )(batch_r, x, sums, sq, cnt128, w2, b2)
    return out


def kernel(x, batch, weight, bias):
    return _impl(x, batch, weight, bias)


# speedup vs baseline: 1.1086x; 1.1086x over previous
import jax
import jax.numpy as jnp
from jax import lax
from jax.experimental import pallas as pl
from jax.experimental.pallas import tpu as pltpu
from jax.experimental.pallas import tpu_sc as plsc

_NUM_GRAPHS = 64
_EPS = 1e-05
_BLOCK = 4000
_N = 100000
_F = 512
_SC_CORES = 2
_SC_SUBCORES = 16
_CHUNK = 96
_FGROUPS = _F // 16


def _sc_stats_body(x_hbm, bnd_hbm, sums_hbm, sq_hbm, bnd_v, xbuf0, xbuf1,
                   acc_s, acc_q, sem0, sem1):
    c = lax.axis_index("c")
    s = lax.axis_index("s")
    w = s * _SC_CORES + c
    pltpu.sync_copy(bnd_hbm, bnd_v)
    zeros = jnp.zeros((16,), jnp.float32)
    for g in range(2):
        for fg in range(_FGROUPS):
            acc_s[g, pl.ds(fg * 16, 16)] = zeros
            acc_q[g, pl.ds(fg * 16, 16)] = zeros
    bv = bnd_v[pl.ds(2 * w, 16)]
    s0 = bv[0]
    s1 = bv[1]
    s2 = bv[2]
    s0a = (s0 // 8) * 8
    nch = (s2 - s0a + _CHUNK - 1) // _CHUNK

    def chunk_r0c(ci):
        r0 = s0a + ci * _CHUNK
        return r0, jnp.minimum(r0, _N - _CHUNK)

    def start(ci, buf, sem):
        _, r0c = chunk_r0c(ci)
        pltpu.async_copy(x_hbm.at[pl.ds(r0c, _CHUNK)], buf, sem)

    def wait(buf, sem):
        pltpu.make_async_copy(x_hbm.at[pl.ds(0, _CHUNK)], buf, sem).wait()

    def compute(ci, buf):
        r0, r0c = chunk_r0c(ci)
        for seg in range(2):
            lo = jnp.maximum(r0, s0 if seg == 0 else s1) - r0c
            hi = jnp.minimum(r0 + _CHUNK, s1 if seg == 0 else s2) - r0c
            for fg in range(_FGROUPS):
                def row_body(r, carry, fg=fg, buf=buf):
                    vs, vq = carry
                    v = buf[r, pl.ds(fg * 16, 16)]
                    return vs + v, vq + v * v

                vs, vq = plsc.parallel_loop(
                    lo, hi, unroll=4, carry=(zeros, zeros))(row_body)
                acc_s[seg, pl.ds(fg * 16, 16)] += vs
                acc_q[seg, pl.ds(fg * 16, 16)] += vq

    start(0, xbuf0, sem0)
    start(1, xbuf1, sem1)

    def outer(k, _):
        c0 = 2 * k
        wait(xbuf0, sem0)
        compute(c0, xbuf0)
        start(c0 + 2, xbuf0, sem0)
        wait(xbuf1, sem1)
        compute(c0 + 1, xbuf1)
        start(c0 + 3, xbuf1, sem1)
        return 0

    lax.fori_loop(0, (nch + 1) // 2, outer, 0)
    wait(xbuf0, sem0)
    wait(xbuf1, sem1)
    pltpu.sync_copy(acc_s, sums_hbm.at[pl.ds(8 * w, 8)])
    pltpu.sync_copy(acc_q, sq_hbm.at[pl.ds(8 * w, 8)])


def _sc_stats(x, bounds, interpret=False):
    mesh = plsc.VectorSubcoreMesh(
        core_axis_name="c", subcore_axis_name="s",
        num_cores=_SC_CORES, num_subcores=_SC_SUBCORES)
    f = x.shape[1]
    return pl.kernel(
        _sc_stats_body,
        out_type=[
            jax.ShapeDtypeStruct((8 * _SC_CORES * _SC_SUBCORES, f), jnp.float32),
            jax.ShapeDtypeStruct((8 * _SC_CORES * _SC_SUBCORES, f), jnp.float32),
        ],
        mesh=mesh,
        scratch_types=[
            pltpu.VMEM((80,), jnp.int32),
            pltpu.VMEM((_CHUNK, f), jnp.float32),
            pltpu.VMEM((_CHUNK, f), jnp.float32),
            pltpu.VMEM((8, f), jnp.float32),
            pltpu.VMEM((8, f), jnp.float32),
            pltpu.SemaphoreType.DMA,
            pltpu.SemaphoreType.DMA,
        ],
        interpret=interpret,
    )(x, bounds)


def _norm_kernel(batch_ref, x_ref, sums_ref, sq_ref, cnt_ref, w_ref, bias_ref,
                 out_ref):
    b = batch_ref[0, 0, :]
    sums = sums_ref[...]
    sq = sq_ref[...]
    cnt = cnt_ref[:, :1]
    mean = sums / jnp.maximum(cnt, 1.0)
    var = (sq - sums * mean) / jnp.maximum(cnt - 1.0, 1.0)
    var = jnp.maximum(var, 0.0)
    scale = w_ref[...] / (jnp.sqrt(var) + _EPS)
    offset = bias_ref[...] - mean * scale
    gids = jax.lax.broadcasted_iota(jnp.int32, (_BLOCK, _NUM_GRAPHS), 1)
    oh = (gids == b[:, None]).astype(jnp.bfloat16)
    tbl = jnp.concatenate([scale, offset], axis=1)
    hi = tbl.astype(jnp.bfloat16)
    lo = (tbl - hi.astype(jnp.float32)).astype(jnp.bfloat16)
    g_hi = jax.lax.dot(oh, hi, preferred_element_type=jnp.float32)
    g_lo = jax.lax.dot(oh, lo, preferred_element_type=jnp.float32)
    g = g_hi + g_lo
    f = x_ref.shape[1]
    out_ref[...] = x_ref[...] * g[:, :f] + g[:, f:]


def _impl(x, batch, weight, bias, interpret=False):
    n, f = x.shape
    nblk = n // _BLOCK
    batch_r = batch.reshape(nblk, 1, _BLOCK)
    w2 = weight.reshape(1, f)
    b2 = bias.reshape(1, f)

    gid = jnp.arange(_NUM_GRAPHS, dtype=jnp.int32)
    starts = jnp.searchsorted(batch, gid).astype(jnp.int32)
    bounds = jnp.concatenate(
        [starts, jnp.full((80 - _NUM_GRAPHS,), n, jnp.int32)])
    cnt = (bounds[1:_NUM_GRAPHS + 1] - bounds[:_NUM_GRAPHS]).astype(jnp.float32)
    cnt128 = jnp.broadcast_to(cnt[:, None], (_NUM_GRAPHS, 128))

    sums_big, sq_big = _sc_stats(x, bounds, interpret=interpret)
    nw = _SC_CORES * _SC_SUBCORES
    sums = sums_big.reshape(nw, 8, f)[:, :2, :].reshape(_NUM_GRAPHS, f)
    sq = sq_big.reshape(nw, 8, f)[:, :2, :].reshape(_NUM_GRAPHS, f)

    out = pl.pallas_call(
        _norm_kernel,
        grid=(nblk,),
        in_specs=[
            pl.BlockSpec((1, 1, _BLOCK), lambda i: (i, 0, 0)),
            pl.BlockSpec((_BLOCK, f), lambda i: (i, 0)),
            pl.BlockSpec((_NUM_GRAPHS, f), lambda i: (0, 0)),
            pl.BlockSpec((_NUM_GRAPHS, f), lambda i: (0, 0)),
            pl.BlockSpec((_NUM_GRAPHS, 128), lambda i: (0, 0)),
            pl.BlockSpec((1, f), lambda i: (0, 0)),
            pl.BlockSpec((1, f), lambda i: (0, 0)),
        ],
        out_specs=pl.BlockSpec((_BLOCK, f), lambda i: (i, 0)),
        out_shape=jax.ShapeDtypeStruct((n, f), jnp.float32),
        interpret=interpret,
    )(batch_r, x, sums, sq, cnt128, w2, b2)
    return out


def kernel(x, batch, weight, bias):
    return _impl(x, batch, weight, bias)


# speedup vs baseline: 2.4682x; 2.2264x over previous
import jax
import jax.numpy as jnp
from jax import lax
from jax.experimental import pallas as pl
from jax.experimental.pallas import tpu as pltpu
from jax.experimental.pallas import tpu_sc as plsc

_NUM_GRAPHS = 64
_EPS = 1e-05
_BLOCK = 4000
_N = 100000
_F = 512
_SC_CORES = 2
_SC_SUBCORES = 16
_CHUNK = 96
_FGROUPS = _F // 16


def _sc_stats_body(x_hbm, bnd_hbm, sums_hbm, sq_hbm, bnd_v, xbuf0, xbuf1,
                   acc_s, acc_q, sem0, sem1):
    c = lax.axis_index("c")
    s = lax.axis_index("s")
    w = s * _SC_CORES + c
    pltpu.sync_copy(bnd_hbm, bnd_v)
    zeros = jnp.zeros((16,), jnp.float32)
    for g in range(2):
        for fg in range(_FGROUPS):
            acc_s[g, pl.ds(fg * 16, 16)] = zeros
            acc_q[g, pl.ds(fg * 16, 16)] = zeros
    bv = bnd_v[pl.ds(2 * w, 16)]
    s0 = bv[0]
    s1 = bv[1]
    s2 = bv[2]
    s0a = (s0 // 8) * 8
    nch = (s2 - s0a + _CHUNK - 1) // _CHUNK

    def chunk_r0c(ci):
        r0 = s0a + ci * _CHUNK
        return r0, jnp.minimum(r0, _N - _CHUNK)

    def start(ci, buf, sem):
        _, r0c = chunk_r0c(ci)
        pltpu.async_copy(x_hbm.at[pl.ds(r0c, _CHUNK)], buf, sem)

    def wait(buf, sem):
        pltpu.make_async_copy(x_hbm.at[pl.ds(0, _CHUNK)], buf, sem).wait()

    _FB = 8

    def compute(ci, buf):
        r0, r0c = chunk_r0c(ci)
        for seg in range(2):
            lo = jnp.maximum(r0, s0 if seg == 0 else s1) - r0c
            hi = jnp.minimum(r0 + _CHUNK, s1 if seg == 0 else s2) - r0c
            for jb in range(_FGROUPS // _FB):
                def row_body(r, carry, jb=jb, buf=buf):
                    vs = carry[:_FB]
                    vq = carry[_FB:]
                    out_s, out_q = [], []
                    for u in range(_FB):
                        v = buf[r, pl.ds((jb * _FB + u) * 16, 16)]
                        out_s.append(vs[u] + v)
                        out_q.append(vq[u] + v * v)
                    return tuple(out_s) + tuple(out_q)

                res = plsc.parallel_loop(
                    lo, hi, unroll=2, carry=(zeros,) * (2 * _FB))(row_body)
                for u in range(_FB):
                    fg = jb * _FB + u
                    acc_s[seg, pl.ds(fg * 16, 16)] += res[u]
                    acc_q[seg, pl.ds(fg * 16, 16)] += res[_FB + u]

    start(0, xbuf0, sem0)
    start(1, xbuf1, sem1)

    def outer(k, _):
        c0 = 2 * k
        wait(xbuf0, sem0)
        compute(c0, xbuf0)
        start(c0 + 2, xbuf0, sem0)
        wait(xbuf1, sem1)
        compute(c0 + 1, xbuf1)
        start(c0 + 3, xbuf1, sem1)
        return 0

    lax.fori_loop(0, (nch + 1) // 2, outer, 0)
    wait(xbuf0, sem0)
    wait(xbuf1, sem1)
    pltpu.sync_copy(acc_s, sums_hbm.at[pl.ds(8 * w, 8)])
    pltpu.sync_copy(acc_q, sq_hbm.at[pl.ds(8 * w, 8)])


def _sc_stats(x, bounds, interpret=False):
    mesh = plsc.VectorSubcoreMesh(
        core_axis_name="c", subcore_axis_name="s",
        num_cores=_SC_CORES, num_subcores=_SC_SUBCORES)
    f = x.shape[1]
    return pl.kernel(
        _sc_stats_body,
        out_type=[
            jax.ShapeDtypeStruct((8 * _SC_CORES * _SC_SUBCORES, f), jnp.float32),
            jax.ShapeDtypeStruct((8 * _SC_CORES * _SC_SUBCORES, f), jnp.float32),
        ],
        mesh=mesh,
        scratch_types=[
            pltpu.VMEM((80,), jnp.int32),
            pltpu.VMEM((_CHUNK, f), jnp.float32),
            pltpu.VMEM((_CHUNK, f), jnp.float32),
            pltpu.VMEM((8, f), jnp.float32),
            pltpu.VMEM((8, f), jnp.float32),
            pltpu.SemaphoreType.DMA,
            pltpu.SemaphoreType.DMA,
        ],
        interpret=interpret,
    )(x, bounds)


def _norm_kernel(batch_ref, x_ref, sums_ref, sq_ref, cnt_ref, w_ref, bias_ref,
                 out_ref):
    b = batch_ref[0, 0, :]
    sums = sums_ref[...]
    sq = sq_ref[...]
    cnt = cnt_ref[:, :1]
    mean = sums / jnp.maximum(cnt, 1.0)
    var = (sq - sums * mean) / jnp.maximum(cnt - 1.0, 1.0)
    var = jnp.maximum(var, 0.0)
    scale = w_ref[...] / (jnp.sqrt(var) + _EPS)
    offset = bias_ref[...] - mean * scale
    gids = jax.lax.broadcasted_iota(jnp.int32, (_BLOCK, _NUM_GRAPHS), 1)
    oh = (gids == b[:, None]).astype(jnp.bfloat16)
    tbl = jnp.concatenate([scale, offset], axis=1)
    hi = tbl.astype(jnp.bfloat16)
    lo = (tbl - hi.astype(jnp.float32)).astype(jnp.bfloat16)
    g_hi = jax.lax.dot(oh, hi, preferred_element_type=jnp.float32)
    g_lo = jax.lax.dot(oh, lo, preferred_element_type=jnp.float32)
    g = g_hi + g_lo
    f = x_ref.shape[1]
    out_ref[...] = x_ref[...] * g[:, :f] + g[:, f:]


def _impl(x, batch, weight, bias, interpret=False):
    n, f = x.shape
    nblk = n // _BLOCK
    batch_r = batch.reshape(nblk, 1, _BLOCK)
    w2 = weight.reshape(1, f)
    b2 = bias.reshape(1, f)

    gid = jnp.arange(_NUM_GRAPHS, dtype=jnp.int32)
    starts = jnp.searchsorted(batch, gid).astype(jnp.int32)
    bounds = jnp.concatenate(
        [starts, jnp.full((80 - _NUM_GRAPHS,), n, jnp.int32)])
    cnt = (bounds[1:_NUM_GRAPHS + 1] - bounds[:_NUM_GRAPHS]).astype(jnp.float32)
    cnt128 = jnp.broadcast_to(cnt[:, None], (_NUM_GRAPHS, 128))

    sums_big, sq_big = _sc_stats(x, bounds, interpret=interpret)
    nw = _SC_CORES * _SC_SUBCORES
    sums = sums_big.reshape(nw, 8, f)[:, :2, :].reshape(_NUM_GRAPHS, f)
    sq = sq_big.reshape(nw, 8, f)[:, :2, :].reshape(_NUM_GRAPHS, f)

    out = pl.pallas_call(
        _norm_kernel,
        grid=(nblk,),
        in_specs=[
            pl.BlockSpec((1, 1, _BLOCK), lambda i: (i, 0, 0)),
            pl.BlockSpec((_BLOCK, f), lambda i: (i, 0)),
            pl.BlockSpec((_NUM_GRAPHS, f), lambda i: (0, 0)),
            pl.BlockSpec((_NUM_GRAPHS, f), lambda i: (0, 0)),
            pl.BlockSpec((_NUM_GRAPHS, 128), lambda i: (0, 0)),
            pl.BlockSpec((1, f), lambda i: (0, 0)),
            pl.BlockSpec((1, f), lambda i: (0, 0)),
        ],
        out_specs=pl.BlockSpec((_BLOCK, f), lambda i: (i, 0)),
        out_shape=jax.ShapeDtypeStruct((n, f), jnp.float32),
        interpret=interpret,
    )(batch_r, x, sums, sq, cnt128, w2, b2)
    return out


def kernel(x, batch, weight, bias):
    return _impl(x, batch, weight, bias)


# speedup vs baseline: 2.5684x; 1.0406x over previous
import jax
import jax.numpy as jnp
from jax import lax
from jax.experimental import pallas as pl
from jax.experimental.pallas import tpu as pltpu
from jax.experimental.pallas import tpu_sc as plsc

_NUM_GRAPHS = 64
_EPS = 1e-05
_BLOCK = 4000
_N = 100000
_F = 512
_SC_CORES = 2
_SC_SUBCORES = 16
_CHUNK = 96
_FGROUPS = _F // 16


def _sc_stats_body(x_hbm, bnd_hbm, sums_hbm, sq_hbm, bnd_v, xbuf0, xbuf1,
                   acc_s, acc_q, sem0, sem1):
    c = lax.axis_index("c")
    s = lax.axis_index("s")
    w = s * _SC_CORES + c
    pltpu.sync_copy(bnd_hbm, bnd_v)
    zeros = jnp.zeros((16,), jnp.float32)
    for g in range(2):
        for fg in range(_FGROUPS):
            acc_s[g, pl.ds(fg * 16, 16)] = zeros
            acc_q[g, pl.ds(fg * 16, 16)] = zeros
    bv = bnd_v[pl.ds(2 * w, 16)]
    s0 = bv[0]
    s1 = bv[1]
    s2 = bv[2]
    s0a = (s0 // 8) * 8
    nch = (s2 - s0a + _CHUNK - 1) // _CHUNK

    def chunk_r0c(ci):
        r0 = s0a + ci * _CHUNK
        return r0, jnp.minimum(r0, _N - _CHUNK)

    def start(ci, buf, sem):
        _, r0c = chunk_r0c(ci)
        pltpu.async_copy(x_hbm.at[pl.ds(r0c, _CHUNK)], buf, sem)

    def wait(buf, sem):
        pltpu.make_async_copy(x_hbm.at[pl.ds(0, _CHUNK)], buf, sem).wait()

    _FB = 8

    def compute(ci, buf):
        r0, r0c = chunk_r0c(ci)
        for seg in range(2):
            lo = jnp.maximum(r0, s0 if seg == 0 else s1) - r0c
            hi = jnp.minimum(r0 + _CHUNK, s1 if seg == 0 else s2) - r0c
            for jb in range(_FGROUPS // _FB):
                def row_body(r, carry, jb=jb, buf=buf):
                    vs = carry[:_FB]
                    vq = carry[_FB:]
                    out_s, out_q = [], []
                    for u in range(_FB):
                        v = buf[r, pl.ds((jb * _FB + u) * 16, 16)]
                        out_s.append(vs[u] + v)
                        out_q.append(vq[u] + v * v)
                    return tuple(out_s) + tuple(out_q)

                res = plsc.parallel_loop(
                    lo, hi, unroll=4, carry=(zeros,) * (2 * _FB))(row_body)
                for u in range(_FB):
                    fg = jb * _FB + u
                    acc_s[seg, pl.ds(fg * 16, 16)] += res[u]
                    acc_q[seg, pl.ds(fg * 16, 16)] += res[_FB + u]

    start(0, xbuf0, sem0)
    start(1, xbuf1, sem1)

    def outer(k, _):
        c0 = 2 * k
        wait(xbuf0, sem0)
        compute(c0, xbuf0)
        start(c0 + 2, xbuf0, sem0)
        wait(xbuf1, sem1)
        compute(c0 + 1, xbuf1)
        start(c0 + 3, xbuf1, sem1)
        return 0

    lax.fori_loop(0, (nch + 1) // 2, outer, 0)
    wait(xbuf0, sem0)
    wait(xbuf1, sem1)
    pltpu.sync_copy(acc_s, sums_hbm.at[pl.ds(8 * w, 8)])
    pltpu.sync_copy(acc_q, sq_hbm.at[pl.ds(8 * w, 8)])


def _sc_stats(x, bounds, interpret=False):
    mesh = plsc.VectorSubcoreMesh(
        core_axis_name="c", subcore_axis_name="s",
        num_cores=_SC_CORES, num_subcores=_SC_SUBCORES)
    f = x.shape[1]
    return pl.kernel(
        _sc_stats_body,
        out_type=[
            jax.ShapeDtypeStruct((8 * _SC_CORES * _SC_SUBCORES, f), jnp.float32),
            jax.ShapeDtypeStruct((8 * _SC_CORES * _SC_SUBCORES, f), jnp.float32),
        ],
        mesh=mesh,
        scratch_types=[
            pltpu.VMEM((80,), jnp.int32),
            pltpu.VMEM((_CHUNK, f), jnp.float32),
            pltpu.VMEM((_CHUNK, f), jnp.float32),
            pltpu.VMEM((8, f), jnp.float32),
            pltpu.VMEM((8, f), jnp.float32),
            pltpu.SemaphoreType.DMA,
            pltpu.SemaphoreType.DMA,
        ],
        interpret=interpret,
    )(x, bounds)


def _norm_kernel(batch_ref, x_ref, sums_ref, sq_ref, cnt_ref, w_ref, bias_ref,
                 out_ref):
    b = batch_ref[0, 0, :]
    sums = sums_ref[...]
    sq = sq_ref[...]
    cnt = cnt_ref[:, :1]
    mean = sums / jnp.maximum(cnt, 1.0)
    var = (sq - sums * mean) / jnp.maximum(cnt - 1.0, 1.0)
    var = jnp.maximum(var, 0.0)
    scale = w_ref[...] / (jnp.sqrt(var) + _EPS)
    offset = bias_ref[...] - mean * scale
    gids = jax.lax.broadcasted_iota(jnp.int32, (_BLOCK, _NUM_GRAPHS), 1)
    oh = (gids == b[:, None]).astype(jnp.bfloat16)
    tbl = jnp.concatenate([scale, offset], axis=1)
    hi = tbl.astype(jnp.bfloat16)
    lo = (tbl - hi.astype(jnp.float32)).astype(jnp.bfloat16)
    oh2 = jnp.concatenate([oh, oh], axis=1)
    hilo = jnp.concatenate([hi, lo], axis=0)
    g = jax.lax.dot(oh2, hilo, preferred_element_type=jnp.float32)
    f = x_ref.shape[1]
    out_ref[...] = x_ref[...] * g[:, :f] + g[:, f:]


def _impl(x, batch, weight, bias, interpret=False):
    n, f = x.shape
    nblk = n // _BLOCK
    batch_r = batch.reshape(nblk, 1, _BLOCK)
    w2 = weight.reshape(1, f)
    b2 = bias.reshape(1, f)

    gid = jnp.arange(_NUM_GRAPHS, dtype=jnp.int32)
    starts = jnp.searchsorted(batch, gid).astype(jnp.int32)
    bounds = jnp.concatenate(
        [starts, jnp.full((80 - _NUM_GRAPHS,), n, jnp.int32)])
    cnt = (bounds[1:_NUM_GRAPHS + 1] - bounds[:_NUM_GRAPHS]).astype(jnp.float32)
    cnt128 = jnp.broadcast_to(cnt[:, None], (_NUM_GRAPHS, 128))

    sums_big, sq_big = _sc_stats(x, bounds, interpret=interpret)
    nw = _SC_CORES * _SC_SUBCORES
    sums = sums_big.reshape(nw, 8, f)[:, :2, :].reshape(_NUM_GRAPHS, f)
    sq = sq_big.reshape(nw, 8, f)[:, :2, :].reshape(_NUM_GRAPHS, f)

    out = pl.pallas_call(
        _norm_kernel,
        grid=(nblk,),
        in_specs=[
            pl.BlockSpec((1, 1, _BLOCK), lambda i: (i, 0, 0)),
            pl.BlockSpec((_BLOCK, f), lambda i: (i, 0)),
            pl.BlockSpec((_NUM_GRAPHS, f), lambda i: (0, 0)),
            pl.BlockSpec((_NUM_GRAPHS, f), lambda i: (0, 0)),
            pl.BlockSpec((_NUM_GRAPHS, 128), lambda i: (0, 0)),
            pl.BlockSpec((1, f), lambda i: (0, 0)),
            pl.BlockSpec((1, f), lambda i: (0, 0)),
        ],
        out_specs=pl.BlockSpec((_BLOCK, f), lambda i: (i, 0)),
        out_shape=jax.ShapeDtypeStruct((n, f), jnp.float32),
        interpret=interpret,
    )(batch_r, x, sums, sq, cnt128, w2, b2)
    return out


def kernel(x, batch, weight, bias):
    return _impl(x, batch, weight, bias)


# speedup vs baseline: 2.5976x; 1.0114x over previous
import jax
import jax.numpy as jnp
from jax import lax
from jax.experimental import pallas as pl
from jax.experimental.pallas import tpu as pltpu
from jax.experimental.pallas import tpu_sc as plsc

_NUM_GRAPHS = 64
_EPS = 1e-05
_BLOCK = 4000
_N = 100000
_F = 512
_SC_CORES = 2
_SC_SUBCORES = 16
_CHUNK = 96
_FGROUPS = _F // 16


def _sc_stats_body(x_hbm, bnd_hbm, sums_hbm, sq_hbm, bnd_v, xbuf0, xbuf1,
                   acc_s, acc_q, sem0, sem1):
    c = lax.axis_index("c")
    s = lax.axis_index("s")
    w = s * _SC_CORES + c
    pltpu.sync_copy(bnd_hbm, bnd_v)
    zeros = jnp.zeros((16,), jnp.float32)
    for g in range(2):
        for fg in range(_FGROUPS):
            acc_s[g, pl.ds(fg * 16, 16)] = zeros
            acc_q[g, pl.ds(fg * 16, 16)] = zeros
    bv = bnd_v[pl.ds(2 * w, 16)]
    s0 = bv[0]
    s1 = bv[1]
    s2 = bv[2]
    s0a = (s0 // 8) * 8
    nch = (s2 - s0a + _CHUNK - 1) // _CHUNK

    def chunk_r0c(ci):
        r0 = s0a + ci * _CHUNK
        return r0, jnp.minimum(r0, _N - _CHUNK)

    def start(ci, buf, sem):
        _, r0c = chunk_r0c(ci)
        pltpu.async_copy(x_hbm.at[pl.ds(r0c, _CHUNK)], buf, sem)

    def wait(buf, sem):
        pltpu.make_async_copy(x_hbm.at[pl.ds(0, _CHUNK)], buf, sem).wait()

    _FB = 8

    def compute(ci, buf):
        r0, r0c = chunk_r0c(ci)
        for seg in range(2):
            lo = jnp.maximum(r0, s0 if seg == 0 else s1) - r0c
            hi = jnp.minimum(r0 + _CHUNK, s1 if seg == 0 else s2) - r0c
            for jb in range(_FGROUPS // _FB):
                def row_body(r, carry, jb=jb, buf=buf):
                    vs = carry[:_FB]
                    vq = carry[_FB:]
                    out_s, out_q = [], []
                    for u in range(_FB):
                        v = buf[r, pl.ds((jb * _FB + u) * 16, 16)]
                        out_s.append(vs[u] + v)
                        out_q.append(vq[u] + v * v)
                    return tuple(out_s) + tuple(out_q)

                res = plsc.parallel_loop(
                    lo, hi, unroll=4, carry=(zeros,) * (2 * _FB))(row_body)
                for u in range(_FB):
                    fg = jb * _FB + u
                    acc_s[seg, pl.ds(fg * 16, 16)] += res[u]
                    acc_q[seg, pl.ds(fg * 16, 16)] += res[_FB + u]

    start(0, xbuf0, sem0)
    start(1, xbuf1, sem1)

    def outer(k, _):
        c0 = 2 * k
        wait(xbuf0, sem0)
        compute(c0, xbuf0)
        start(c0 + 2, xbuf0, sem0)
        wait(xbuf1, sem1)
        compute(c0 + 1, xbuf1)
        start(c0 + 3, xbuf1, sem1)
        return 0

    lax.fori_loop(0, (nch + 1) // 2, outer, 0)
    wait(xbuf0, sem0)
    wait(xbuf1, sem1)
    pltpu.sync_copy(acc_s, sums_hbm.at[pl.ds(8 * w, 8)])
    pltpu.sync_copy(acc_q, sq_hbm.at[pl.ds(8 * w, 8)])


def _sc_stats(x, bounds, interpret=False):
    mesh = plsc.VectorSubcoreMesh(
        core_axis_name="c", subcore_axis_name="s",
        num_cores=_SC_CORES, num_subcores=_SC_SUBCORES)
    f = x.shape[1]
    return pl.kernel(
        _sc_stats_body,
        out_type=[
            jax.ShapeDtypeStruct((8 * _SC_CORES * _SC_SUBCORES, f), jnp.float32),
            jax.ShapeDtypeStruct((8 * _SC_CORES * _SC_SUBCORES, f), jnp.float32),
        ],
        mesh=mesh,
        scratch_types=[
            pltpu.VMEM((80,), jnp.int32),
            pltpu.VMEM((_CHUNK, f), jnp.float32),
            pltpu.VMEM((_CHUNK, f), jnp.float32),
            pltpu.VMEM((8, f), jnp.float32),
            pltpu.VMEM((8, f), jnp.float32),
            pltpu.SemaphoreType.DMA,
            pltpu.SemaphoreType.DMA,
        ],
        interpret=interpret,
    )(x, bounds)


def _norm_kernel(batch_ref, x_ref, sums_ref, sq_ref, cnt_ref, w_ref, bias_ref,
                 out_ref):
    b = batch_ref[0, 0, :]
    sums = sums_ref[...]
    sq = sq_ref[...]
    cnt = cnt_ref[:, :1]
    mean = sums / jnp.maximum(cnt, 1.0)
    var = (sq - sums * mean) / jnp.maximum(cnt - 1.0, 1.0)
    var = jnp.maximum(var, 0.0)
    scale = w_ref[...] / (jnp.sqrt(var) + _EPS)
    offset = bias_ref[...] - mean * scale
    gids = jax.lax.broadcasted_iota(jnp.int32, (_BLOCK, _NUM_GRAPHS), 1)
    oh = (gids == b[:, None]).astype(jnp.bfloat16)
    tbl = jnp.concatenate([scale, offset], axis=1).astype(jnp.bfloat16)
    g = jax.lax.dot(oh, tbl, preferred_element_type=jnp.float32)
    f = x_ref.shape[1]
    out_ref[...] = x_ref[...] * g[:, :f] + g[:, f:]


def _impl(x, batch, weight, bias, interpret=False):
    n, f = x.shape
    nblk = n // _BLOCK
    batch_r = batch.reshape(nblk, 1, _BLOCK)
    w2 = weight.reshape(1, f)
    b2 = bias.reshape(1, f)

    gid = jnp.arange(_NUM_GRAPHS, dtype=jnp.int32)
    starts = jnp.searchsorted(batch, gid).astype(jnp.int32)
    bounds = jnp.concatenate(
        [starts, jnp.full((80 - _NUM_GRAPHS,), n, jnp.int32)])
    cnt = (bounds[1:_NUM_GRAPHS + 1] - bounds[:_NUM_GRAPHS]).astype(jnp.float32)
    cnt128 = jnp.broadcast_to(cnt[:, None], (_NUM_GRAPHS, 128))

    sums_big, sq_big = _sc_stats(x, bounds, interpret=interpret)
    nw = _SC_CORES * _SC_SUBCORES
    sums = sums_big.reshape(nw, 8, f)[:, :2, :].reshape(_NUM_GRAPHS, f)
    sq = sq_big.reshape(nw, 8, f)[:, :2, :].reshape(_NUM_GRAPHS, f)

    out = pl.pallas_call(
        _norm_kernel,
        grid=(nblk,),
        in_specs=[
            pl.BlockSpec((1, 1, _BLOCK), lambda i: (i, 0, 0)),
            pl.BlockSpec((_BLOCK, f), lambda i: (i, 0)),
            pl.BlockSpec((_NUM_GRAPHS, f), lambda i: (0, 0)),
            pl.BlockSpec((_NUM_GRAPHS, f), lambda i: (0, 0)),
            pl.BlockSpec((_NUM_GRAPHS, 128), lambda i: (0, 0)),
            pl.BlockSpec((1, f), lambda i: (0, 0)),
            pl.BlockSpec((1, f), lambda i: (0, 0)),
        ],
        out_specs=pl.BlockSpec((_BLOCK, f), lambda i: (i, 0)),
        out_shape=jax.ShapeDtypeStruct((n, f), jnp.float32),
        interpret=interpret,
    )(batch_r, x, sums, sq, cnt128, w2, b2)
    return out


def kernel(x, batch, weight, bias):
    return _impl(x, batch, weight, bias)
